# Initial kernel scaffold; baseline (speedup 1.0000x reference)
import jax
import jax.numpy as jnp
from jax.experimental import pallas as pl


def kernel(node_features, edge_index, move_nodes, move_mask, t, Wt1, bt1, Wt2, bt2, We1, be1, We2, be2, Wg, bg, Wd1, bd1, Wd2, bd2, Wd3, bd3):
    raise NotImplementedError("write your pallas kernel here")


# speedup vs baseline: 55.0685x; 55.0685x over previous
import functools

import jax
import jax.numpy as jnp
import numpy as np
from jax import lax
from jax.experimental import pallas as pl
from jax.experimental.pallas import tpu as pltpu
from jax.experimental.pallas import tpu_sc as plsc

HD = 128
NF = 128
NL = 6
B, N, E, M = 2, 10000, 160000, 2048

NP = 10240
NTILES = 16
ROWS_PT = NP // NTILES
CH = 128
CPT = -(-E // (NTILES * CH))
EPT = CPT * CH
EPAD = NTILES * EPT
MF = M * 4
MCH = MF // NTILES // CH


def _silu(x):
    return x * (1.0 / (1.0 + jnp.exp(-x)))


_BLK = 1024


def _embed_body(te_ref, nf_ref, wt1, bt1, wt2, bt2, we1, be1, we2, be2,
                out_ref):
    te = te_ref[0]
    t1 = _silu(jnp.dot(te, wt1[...], preferred_element_type=jnp.float32)
               + bt1[...])
    temb = jnp.dot(t1, wt2[...], preferred_element_type=jnp.float32) + bt2[...]
    nf = nf_ref[0]
    h1 = _silu(jnp.dot(nf, we1[...], preferred_element_type=jnp.float32)
               + be1[...])
    h = jnp.dot(h1, we2[...], preferred_element_type=jnp.float32) + be2[...]
    out_ref[0] = h + temb


def _layer_body(h_ref, agg_ref, wa, wb, bgr, out_ref):
    h = h_ref[0]
    a = agg_ref[0]
    z = (jnp.dot(h, wa[...], preferred_element_type=jnp.float32)
         + jnp.dot(a, wb[...], preferred_element_type=jnp.float32) + bgr[...])
    out_ref[0] = h + _silu(z)


def _score_body(hm_ref, mask_ref, w1, b1, w2, b2, w3, b3, out_ref):
    x = hm_ref[0]
    s = _silu(jnp.dot(x, w1[...], preferred_element_type=jnp.float32) + b1[...])
    s = _silu(jnp.dot(s, w2[...], preferred_element_type=jnp.float32) + b2[...])
    sc = jnp.dot(s, w3[...], preferred_element_type=jnp.float32) + b3[...]
    m = mask_ref[0]
    out_ref[0] = jnp.where(m != 0, sc, -jnp.inf)


def _full(shape):
    return pl.BlockSpec(shape, lambda b, i: tuple(0 for _ in shape))


def _embed_tc(te3, nf_pad, wt1, bt1, wt2, bt2, we1, be1, we2, be2):
    grid = (B, NP // _BLK)
    return pl.pallas_call(
        _embed_body,
        grid=grid,
        in_specs=[
            pl.BlockSpec((1, 1, 16), lambda b, i: (b, 0, 0)),
            pl.BlockSpec((1, _BLK, NF), lambda b, i: (b, i, 0)),
            _full((16, HD)), _full((1, HD)),
            _full((HD, HD)), _full((1, HD)),
            _full((NF, HD)), _full((1, HD)),
            _full((HD, HD)), _full((1, HD)),
        ],
        out_specs=pl.BlockSpec((1, _BLK, HD), lambda b, i: (b, i, 0)),
        out_shape=jax.ShapeDtypeStruct((B, NP, HD), jnp.float32),
    )(te3, nf_pad, wt1, bt1, wt2, bt2, we1, be1, we2, be2)


def _layer_tc(h, agg, wa, wb, bgr):
    grid = (B, NP // _BLK)
    return pl.pallas_call(
        _layer_body,
        grid=grid,
        in_specs=[
            pl.BlockSpec((1, _BLK, HD), lambda b, i: (b, i, 0)),
            pl.BlockSpec((1, _BLK, HD), lambda b, i: (b, i, 0)),
            _full((HD, HD)), _full((HD, HD)), _full((1, HD)),
        ],
        out_specs=pl.BlockSpec((1, _BLK, HD), lambda b, i: (b, i, 0)),
        out_shape=jax.ShapeDtypeStruct((B, NP, HD), jnp.float32),
    )(h, agg, wa, wb, bgr)


def _score_tc(hm, mask3, w1, b1, w2, b2, w3, b3):
    grid = (B,)
    return pl.pallas_call(
        _score_body,
        grid=grid,
        in_specs=[
            pl.BlockSpec((1, M, 4 * HD), lambda b: (b, 0, 0)),
            pl.BlockSpec((1, M, 1), lambda b: (b, 0, 0)),
            pl.BlockSpec((4 * HD, HD), lambda b: (0, 0)),
            pl.BlockSpec((1, HD), lambda b: (0, 0)),
            pl.BlockSpec((HD, HD), lambda b: (0, 0)),
            pl.BlockSpec((1, HD), lambda b: (0, 0)),
            pl.BlockSpec((HD, 1), lambda b: (0, 0)),
            pl.BlockSpec((1, 1), lambda b: (0, 0)),
        ],
        out_specs=pl.BlockSpec((1, M, 1), lambda b: (b, 0, 0)),
        out_shape=jax.ShapeDtypeStruct((B, M, 1), jnp.float32),
    )(hm, mask3, w1, b1, w2, b2, w3, b3)


_SC_MESH = plsc.VectorSubcoreMesh(core_axis_name="c", subcore_axis_name="s")


@functools.partial(
    pl.kernel,
    out_type=jax.ShapeDtypeStruct((B * NP, HD), jnp.float32),
    mesh=_SC_MESH,
    scratch_types=[
        pltpu.VMEM((CPT, CH), jnp.int32),
        pltpu.VMEM((CPT, CH), jnp.int32),
        pltpu.VMEM((CH, HD), jnp.float32),
        pltpu.VMEM_SHARED((NP, HD), jnp.float32),
        pltpu.SemaphoreType.DMA,
    ],
)
def _agg_sc(h_hbm, src_hbm, dst_hbm, zeros_hbm, out_hbm,
            src_v, dst_v, rows_v, agg_sh, sem):
    c = lax.axis_index("c")
    s = lax.axis_index("s")
    w = c * NTILES + s
    pltpu.sync_copy(zeros_hbm, rows_v)
    for k in range(ROWS_PT // CH):
        pltpu.sync_copy(rows_v, agg_sh.at[pl.ds(s * ROWS_PT + k * CH, CH)])
    pltpu.sync_copy(src_hbm.at[w], src_v)
    pltpu.sync_copy(dst_hbm.at[s], dst_v)
    plsc.subcore_barrier()

    def edge_chunk(j, carry):
        pltpu.async_copy(h_hbm.at[src_v.at[j]], rows_v, sem).wait()
        pltpu.sync_copy(rows_v, agg_sh.at[dst_v.at[j]], add=True)
        return carry

    lax.fori_loop(0, CPT, edge_chunk, 0)
    plsc.subcore_barrier()
    base = c * NP + s * ROWS_PT
    for k in range(ROWS_PT // CH):
        pltpu.sync_copy(agg_sh.at[pl.ds(s * ROWS_PT + k * CH, CH)], rows_v)
        pltpu.sync_copy(rows_v, out_hbm.at[pl.ds(base + k * CH, CH)])


@functools.partial(
    pl.kernel,
    out_type=jax.ShapeDtypeStruct((B * MF, HD), jnp.float32),
    mesh=_SC_MESH,
    scratch_types=[
        pltpu.VMEM((MCH, CH), jnp.int32),
        pltpu.VMEM((CH, HD), jnp.float32),
        pltpu.SemaphoreType.DMA,
    ],
)
def _gather_sc(h_hbm, idx_hbm, out_hbm, idx_v, rows_v, sem):
    c = lax.axis_index("c")
    s = lax.axis_index("s")
    w = c * NTILES + s
    pltpu.sync_copy(idx_hbm.at[w], idx_v)
    base = c * MF + s * (MCH * CH)
    for j in range(MCH):
        pltpu.async_copy(h_hbm.at[idx_v.at[j]], rows_v, sem).wait()
        pltpu.sync_copy(rows_v, out_hbm.at[pl.ds(base + j * CH, CH)])


def kernel(node_features, edge_index, move_nodes, move_mask, t,
           Wt1, bt1, Wt2, bt2, We1, be1, We2, be2, Wg, bg,
           Wd1, bd1, Wd2, bd2, Wd3, bd3):
    freqs = jnp.exp(jnp.arange(0, 8, dtype=jnp.float32)
                    * (-np.log(10000.0) / 8.0))
    args = t[:, None] * freqs
    te3 = jnp.concatenate([jnp.sin(args), jnp.cos(args)],
                          axis=-1).reshape(B, 1, 16)

    nf_pad = jnp.pad(node_features, ((0, 0), (0, NP - N), (0, 0)))

    boff = (jnp.arange(B, dtype=jnp.int32) * NP)[:, None]
    src = jnp.concatenate(
        [edge_index[0], jnp.zeros((EPAD - E,), jnp.int32)])
    dst = jnp.concatenate(
        [edge_index[1], jnp.full((EPAD - E,), N, jnp.int32)])
    src2 = (src[None, :] + boff).reshape(B * NTILES, CPT, CH)
    dstt = dst.reshape(NTILES, CPT, CH)
    zeros_rows = jnp.zeros((CH, HD), jnp.float32)

    midx = (jnp.clip(move_nodes, 0, N - 1).reshape(B, MF)
            + boff).reshape(B * NTILES, MCH, CH)
    mask3 = move_mask.astype(jnp.int32).reshape(B, M, 1)

    b2 = lambda v: v.reshape(1, -1)

    h = _embed_tc(te3, nf_pad, Wt1, b2(bt1), Wt2, b2(bt2),
                  We1, b2(be1), We2, b2(be2))
    for l in range(NL):
        hf = h.reshape(B * NP, HD)
        agg = _agg_sc(hf, src2, dstt, zeros_rows).reshape(B, NP, HD)
        h = _layer_tc(h, agg, Wg[l, :HD, :], Wg[l, HD:, :], b2(bg[l]))

    hm = _gather_sc(h.reshape(B * NP, HD), midx).reshape(B, M, 4 * HD)
    out = _score_tc(hm, mask3, Wd1, b2(bd1), Wd2, b2(bd2), Wd3,
                    bd3.reshape(1, 1))
    return out.reshape(B, M)
